# baseline (speedup 1.0000x reference)
import functools

import jax
import jax.numpy as jnp
from jax import lax
from jax.experimental import pallas as pl
from jax.experimental.pallas import tpu as pltpu
from jax.experimental.pallas import tpu_sc as plsc

B = 16384
F = 100
V = 1000001

NC = 2
NS = 16
L = 16
NW = NC * NS

ROWS_PER_W = B // NW
CHUNK = 128
NCHUNK = ROWS_PER_W // CHUNK
GRP = CHUNK // L
CE = CHUNK * F

NPIECE_FULL = V // CE
TAIL_OFF = NPIECE_FULL * CE
TAIL = V - TAIL_OFF


def kernel(index, value, emb_table, bias):
    idx_t = index.astype(jnp.int32).T
    val_t = value.T
    tab_t = emb_table.T

    mesh = plsc.VectorSubcoreMesh(core_axis_name="c", subcore_axis_name="s")

    @functools.partial(
        pl.kernel,
        mesh=mesh,
        out_type=jax.ShapeDtypeStruct((B,), jnp.float32),
        compiler_params=pltpu.CompilerParams(needs_layout_passes=False),
        scratch_types=[
            pltpu.VMEM((F, CHUNK), jnp.int32),
            pltpu.VMEM((F, CHUNK), jnp.float32),
            pltpu.VMEM((F, CHUNK), jnp.float32),
            pltpu.VMEM((CE,), jnp.float32),
            pltpu.VMEM((CHUNK,), jnp.float32),
            pltpu.VMEM((L,), jnp.float32),
            pltpu.VMEM_SHARED((V,), jnp.float32),
            pltpu.SemaphoreType.DMA,
        ],
    )
    def k(idx_hbm, val_hbm, tab_hbm, bias_hbm, out_hbm,
          idx_v, val_v, gat_v, stage_v, out_v, bias_v, tab_sh, sem):
        cid = lax.axis_index("c")
        sid = lax.axis_index("s")
        wid = sid * NC + cid

        tab_flat = tab_hbm.at[0]

        with jax.named_scope("stage_table"):
            for j in range((NPIECE_FULL + NS - 1) // NS):
                p = sid + j * NS

                @pl.when(p < NPIECE_FULL)
                def _():
                    off = p * CE
                    pltpu.sync_copy(tab_flat.at[pl.ds(off, CE)], stage_v)
                    pltpu.sync_copy(stage_v, tab_sh.at[pl.ds(off, CE)])

            @pl.when(sid == 0)
            def _():
                pltpu.sync_copy(tab_flat.at[pl.ds(TAIL_OFF, TAIL)],
                                stage_v.at[pl.ds(0, TAIL)])
                pltpu.sync_copy(stage_v.at[pl.ds(0, TAIL)],
                                tab_sh.at[pl.ds(TAIL_OFF, TAIL)])

            pltpu.sync_copy(bias_hbm, bias_v.at[pl.ds(0, 1)])
            plsc.subcore_barrier()

        bias_s = bias_v[pl.ds(0, L)][0]

        row_base = wid * ROWS_PER_W
        for kc in range(NCHUNK):
            r0 = row_base + kc * CHUNK
            with jax.named_scope("in_copy"):
                pltpu.sync_copy(idx_hbm.at[:, pl.ds(r0, CHUNK)], idx_v)
                pltpu.sync_copy(val_hbm.at[:, pl.ds(r0, CHUNK)], val_v)

            def fire(f, _):
                pltpu.make_async_copy(
                    tab_sh.at[idx_v.at[f]], gat_v.at[f], sem).start()
                return ()

            with jax.named_scope("fire"):
                lax.fori_loop(0, F, fire, (), unroll=4)

            def drain(f, _):
                pltpu.make_async_copy(
                    tab_sh.at[idx_v.at[f]], gat_v.at[f], sem).wait()
                return ()

            with jax.named_scope("drain"):
                lax.fori_loop(0, F, drain, (), unroll=4)

            def body(f, accs):
                new = []
                for g in range(GRP):
                    gv = gat_v[f, pl.ds(g * L, L)]
                    vv = val_v[f, pl.ds(g * L, L)]
                    new.append(accs[g] + gv * vv)
                return tuple(new)

            with jax.named_scope("compute"):
                accs = lax.fori_loop(
                    0, F, body,
                    tuple(jnp.zeros((L,), jnp.float32) for _ in range(GRP)),
                    unroll=2)
                for g in range(GRP):
                    out_v[pl.ds(g * L, L)] = accs[g] + bias_s

                pltpu.sync_copy(out_v, out_hbm.at[pl.ds(r0, CHUNK)])

    return k(idx_t, val_t, tab_t, bias)


# speedup vs baseline: 1.2137x; 1.2137x over previous
import functools

import jax
import jax.numpy as jnp
from jax import lax
from jax.experimental import pallas as pl
from jax.experimental.pallas import tpu as pltpu
from jax.experimental.pallas import tpu_sc as plsc

B = 16384
F = 100
V = 1000001

NC = 2
NS = 16
L = 16
NW = NC * NS

ROWS_PER_W = B // NW
CHUNK = 128
NCHUNK = ROWS_PER_W // CHUNK
GRP = CHUNK // L

PIECE = 6400
NPIECE_FULL = V // PIECE
NPJ = (NPIECE_FULL + NS - 1) // NS
TAIL_OFF = NPIECE_FULL * PIECE
TAIL = V - TAIL_OFF


def kernel(index, value, emb_table, bias):
    idx_t = index.astype(jnp.int32).T
    val_t = value.T
    tab_t = emb_table.T

    mesh = plsc.VectorSubcoreMesh(core_axis_name="c", subcore_axis_name="s")

    @functools.partial(
        pl.kernel,
        mesh=mesh,
        out_type=jax.ShapeDtypeStruct((B,), jnp.float32),
        compiler_params=pltpu.CompilerParams(needs_layout_passes=False),
        scratch_types=[
            pltpu.VMEM((F, CHUNK), jnp.int32),
            pltpu.VMEM((F, CHUNK), jnp.int32),
            pltpu.VMEM((F, CHUNK), jnp.float32),
            pltpu.VMEM((F, CHUNK), jnp.float32),
            pltpu.VMEM((CHUNK,), jnp.float32),
            pltpu.VMEM((L,), jnp.float32),
            pltpu.VMEM_SHARED((V,), jnp.float32),
            pltpu.VMEM((PIECE,), jnp.float32),
            pltpu.VMEM((PIECE,), jnp.float32),
            pltpu.SemaphoreType.DMA,
            pltpu.SemaphoreType.DMA,
            pltpu.SemaphoreType.DMA,
            pltpu.SemaphoreType.DMA,
        ],
    )
    def k(idx_hbm, val_hbm, tab_hbm, bias_hbm, out_hbm,
          idx0, idx1, val_v, gat_v, out_v, bias_v, tab_sh,
          stg0, stg1, sem_st, sem_idx, sem_val, sem_g):
        cid = lax.axis_index("c")
        sid = lax.axis_index("s")
        wid = sid * NC + cid

        idx_b = [idx0, idx1]
        stgs = [stg0, stg1]

        tab_flat = tab_hbm.at[0]
        row_base = wid * ROWS_PER_W

        def idx_copy(kc):
            r0 = row_base + kc * CHUNK
            return pltpu.make_async_copy(
                idx_hbm.at[:, pl.ds(r0, CHUNK)], idx_b[kc % 2], sem_idx)

        def val_copy(kc):
            r0 = row_base + kc * CHUNK
            return pltpu.make_async_copy(
                val_hbm.at[:, pl.ds(r0, CHUNK)], val_v, sem_val)

        def stg_in(j, p):
            return pltpu.make_async_copy(
                tab_flat.at[pl.ds(p * PIECE, PIECE)], stgs[j % 2], sem_st)

        with jax.named_scope("stage_table"):
            idx_copy(0).start()
            val_copy(0).start()

            @pl.when(sid < NPIECE_FULL)
            def _():
                stg_in(0, sid).start()

            for j in range(NPJ):
                p = sid + j * NS
                pnext = sid + (j + 1) * NS
                if j + 1 < NPJ:
                    @pl.when(pnext < NPIECE_FULL)
                    def _(j=j, pnext=pnext):
                        stg_in(j + 1, pnext).start()

                @pl.when(p < NPIECE_FULL)
                def _(j=j, p=p):
                    stg_in(j, p).wait()
                    pltpu.sync_copy(stgs[j % 2],
                                    tab_sh.at[pl.ds(p * PIECE, PIECE)])

            @pl.when(sid == 0)
            def _():
                pltpu.sync_copy(tab_flat.at[pl.ds(TAIL_OFF, TAIL)],
                                stg0.at[pl.ds(0, TAIL)])
                pltpu.sync_copy(stg0.at[pl.ds(0, TAIL)],
                                tab_sh.at[pl.ds(TAIL_OFF, TAIL)])

            pltpu.sync_copy(bias_hbm, bias_v.at[pl.ds(0, 1)])
            plsc.subcore_barrier()

        bias_s = bias_v[pl.ds(0, L)][0]

        for kc in range(NCHUNK):
            r0 = row_base + kc * CHUNK
            idx_v = idx_b[kc % 2]

            with jax.named_scope("wait_in"):
                idx_copy(kc).wait()
                if kc + 1 < NCHUNK:
                    idx_copy(kc + 1).start()

            def fire(f, _):
                pltpu.make_async_copy(
                    tab_sh.at[idx_v.at[f]], gat_v.at[f], sem_g).start()
                return ()

            with jax.named_scope("fire"):
                lax.fori_loop(0, F, fire, (), unroll=4)

            def drain(f, _):
                pltpu.make_async_copy(
                    tab_sh.at[idx_v.at[f]], gat_v.at[f], sem_g).wait()
                return ()

            with jax.named_scope("drain"):
                lax.fori_loop(0, F, drain, (), unroll=4)
                val_copy(kc).wait()

            def body(f, accs):
                new = []
                for g in range(GRP):
                    gv = gat_v[f, pl.ds(g * L, L)]
                    vv = val_v[f, pl.ds(g * L, L)]
                    new.append(accs[g] + gv * vv)
                return tuple(new)

            with jax.named_scope("compute"):
                accs = lax.fori_loop(
                    0, F, body,
                    tuple(jnp.zeros((L,), jnp.float32) for _ in range(GRP)),
                    unroll=2)
                for g in range(GRP):
                    out_v[pl.ds(g * L, L)] = accs[g] + bias_s

                pltpu.sync_copy(out_v, out_hbm.at[pl.ds(r0, CHUNK)])
                if kc + 1 < NCHUNK:
                    val_copy(kc + 1).start()

    return k(idx_t, val_t, tab_t, bias)


# speedup vs baseline: 1.2791x; 1.0539x over previous
import functools

import jax
import jax.numpy as jnp
from jax import lax
from jax.experimental import pallas as pl
from jax.experimental.pallas import tpu as pltpu
from jax.experimental.pallas import tpu_sc as plsc

B = 16384
F = 100
V = 1000001

NC = 2
NS = 16
L = 16
NW = NC * NS

ROWS_PER_W = B // NW
CHUNK = 128
NCHUNK = ROWS_PER_W // CHUNK
GRP = CHUNK // L

PIECE = 6400
NPIECE_FULL = V // PIECE
NPJ = (NPIECE_FULL + NS - 1) // NS
TAIL_OFF = NPIECE_FULL * PIECE
TAIL = V - TAIL_OFF


def kernel(index, value, emb_table, bias):
    idx_t = index.astype(jnp.int32).T
    val_t = value.T
    tab_t = emb_table.T

    mesh = plsc.VectorSubcoreMesh(core_axis_name="c", subcore_axis_name="s")

    @functools.partial(
        pl.kernel,
        mesh=mesh,
        out_type=jax.ShapeDtypeStruct((B,), jnp.float32),
        compiler_params=pltpu.CompilerParams(needs_layout_passes=False),
        scratch_types=[
            pltpu.VMEM((F, CHUNK), jnp.int32),
            pltpu.VMEM((F, CHUNK), jnp.int32),
            pltpu.VMEM((F, CHUNK), jnp.float32),
            pltpu.VMEM((F, CHUNK), jnp.float32),
            pltpu.VMEM((CHUNK,), jnp.float32),
            pltpu.VMEM((L,), jnp.float32),
            pltpu.VMEM_SHARED((V,), jnp.float32),
            pltpu.VMEM((PIECE,), jnp.float32),
            pltpu.VMEM((PIECE,), jnp.float32),
            pltpu.SemaphoreType.DMA,
            pltpu.SemaphoreType.DMA,
            pltpu.SemaphoreType.DMA,
            pltpu.SemaphoreType.DMA,
        ],
    )
    def k(idx_hbm, val_hbm, tab_hbm, bias_hbm, out_hbm,
          idx0, idx1, val_v, gat_v, out_v, bias_v, tab_sh,
          stg0, stg1, sem_st, sem_idx, sem_val, sem_g):
        cid = lax.axis_index("c")
        sid = lax.axis_index("s")
        wid = sid * NC + cid

        idx_b = [idx0, idx1]
        stgs = [stg0, stg1]

        tab_flat = tab_hbm.at[0]
        row_base = wid * ROWS_PER_W

        def idx_copy(kc):
            r0 = row_base + kc * CHUNK
            return pltpu.make_async_copy(
                idx_hbm.at[:, pl.ds(r0, CHUNK)], idx_b[kc % 2], sem_idx)

        def val_copy(kc):
            r0 = row_base + kc * CHUNK
            return pltpu.make_async_copy(
                val_hbm.at[:, pl.ds(r0, CHUNK)], val_v, sem_val)

        def stg_in(j, p):
            return pltpu.make_async_copy(
                tab_flat.at[pl.ds(p * PIECE, PIECE)], stgs[j % 2], sem_st)

        with jax.named_scope("stage_table"):
            idx_copy(0).start()
            val_copy(0).start()

            @pl.when(sid < NPIECE_FULL)
            def _():
                stg_in(0, sid).start()

            for j in range(NPJ):
                p = sid + j * NS
                pnext = sid + (j + 1) * NS
                if j + 1 < NPJ:
                    @pl.when(pnext < NPIECE_FULL)
                    def _(j=j, pnext=pnext):
                        stg_in(j + 1, pnext).start()

                @pl.when(p < NPIECE_FULL)
                def _(j=j, p=p):
                    stg_in(j, p).wait()
                    pltpu.sync_copy(stgs[j % 2],
                                    tab_sh.at[pl.ds(p * PIECE, PIECE)])

            @pl.when(sid == 0)
            def _():
                pltpu.sync_copy(tab_flat.at[pl.ds(TAIL_OFF, TAIL)],
                                stg0.at[pl.ds(0, TAIL)])
                pltpu.sync_copy(stg0.at[pl.ds(0, TAIL)],
                                tab_sh.at[pl.ds(TAIL_OFF, TAIL)])

            pltpu.sync_copy(bias_hbm, bias_v.at[pl.ds(0, 1)])
            plsc.subcore_barrier()

        bias_s = bias_v[pl.ds(0, L)][0]

        for kc in range(NCHUNK):
            r0 = row_base + kc * CHUNK
            idx_v = idx_b[kc % 2]

            with jax.named_scope("wait_in"):
                idx_copy(kc).wait()
                if kc + 1 < NCHUNK:
                    idx_copy(kc + 1).start()

            def fire(f, _):
                pltpu.make_async_copy(
                    tab_sh.at[idx_v.at[f]], gat_v.at[f], sem_g).start()
                return ()

            with jax.named_scope("fire"):
                lax.fori_loop(0, F, fire, (), unroll=8)
                val_copy(kc).wait()

            def body(f, accs):
                pltpu.make_async_copy(
                    tab_sh.at[idx_v.at[f]], gat_v.at[f], sem_g).wait()
                new = []
                for g in range(GRP):
                    gv = gat_v[f, pl.ds(g * L, L)]
                    vv = val_v[f, pl.ds(g * L, L)]
                    new.append(accs[g] + gv * vv)
                return tuple(new)

            with jax.named_scope("drain_compute"):
                accs = lax.fori_loop(
                    0, F, body,
                    tuple(jnp.zeros((L,), jnp.float32) for _ in range(GRP)),
                    unroll=2)
                for g in range(GRP):
                    out_v[pl.ds(g * L, L)] = accs[g] + bias_s

                pltpu.sync_copy(out_v, out_hbm.at[pl.ds(r0, CHUNK)])
                if kc + 1 < NCHUNK:
                    val_copy(kc + 1).start()

    return k(idx_t, val_t, tab_t, bias)
